# baseline (device time: 501530 ns/iter reference)
import jax
import jax.numpy as jnp
from jax import lax
from jax.experimental import pallas as pl
from jax.experimental.pallas import tpu as pltpu

Z = 4
T = 4096
D = 2048
V_SHARD = 8192
CHUNK = T // Z
N_STEPS = 2 * (Z - 1)


def _allreduce_z(partial):

    def body(p_ref, out_ref, send_buf, recv_buf, send_sems, recv_sems,
             credit_sem):
        my_x = lax.axis_index("x")
        my_y = lax.axis_index("y")
        my_z = lax.axis_index("z")
        next_id = (my_x, my_y, (my_z + 1) % Z)
        prev_id = (my_x, my_y, (my_z - 1) % Z)

        barrier_sem = pltpu.get_barrier_semaphore()
        for nbr in (prev_id, next_id):
            pl.semaphore_signal(
                barrier_sem, inc=1,
                device_id=nbr, device_id_type=pl.DeviceIdType.MESH,
            )
        pl.semaphore_wait(barrier_sem, 2)

        out_ref[...] = p_ref[...]

        for s in range(N_STEPS):
            slot = s % 2
            if s < Z - 1:
                send_idx = (my_z - s) % Z
            else:
                send_idx = (my_z + 1 - (s - (Z - 1))) % Z
            recv_idx = (send_idx - 1) % Z

            send_buf[slot] = out_ref[pl.ds(send_idx * CHUNK, CHUNK), :]

            if s >= 2:
                pl.semaphore_wait(credit_sem, 1)

            rdma = pltpu.make_async_remote_copy(
                src_ref=send_buf.at[slot],
                dst_ref=recv_buf.at[slot],
                send_sem=send_sems.at[slot],
                recv_sem=recv_sems.at[slot],
                device_id=next_id,
                device_id_type=pl.DeviceIdType.MESH,
            )
            rdma.start()
            rdma.wait()

            if s < Z - 1:
                out_ref[pl.ds(recv_idx * CHUNK, CHUNK), :] += recv_buf[slot]
            else:
                out_ref[pl.ds(recv_idx * CHUNK, CHUNK), :] = recv_buf[slot]

            if s < N_STEPS - 2:
                pl.semaphore_signal(
                    credit_sem, inc=1,
                    device_id=prev_id, device_id_type=pl.DeviceIdType.MESH,
                )

    return pl.pallas_call(
        body,
        out_shape=jax.ShapeDtypeStruct((T, D), jnp.bfloat16),
        in_specs=[pl.BlockSpec(memory_space=pltpu.VMEM)],
        out_specs=pl.BlockSpec(memory_space=pltpu.VMEM),
        scratch_shapes=[
            pltpu.VMEM((2, CHUNK, D), jnp.bfloat16),
            pltpu.VMEM((2, CHUNK, D), jnp.bfloat16),
            pltpu.SemaphoreType.DMA((2,)),
            pltpu.SemaphoreType.DMA((2,)),
            pltpu.SemaphoreType.REGULAR,
        ],
        compiler_params=pltpu.CompilerParams(collective_id=0),
    )(partial)


def kernel(ids, E):
    z = lax.axis_index("z")
    local = ids - z * V_SHARD
    owned = (local >= 0) & (local < V_SHARD)
    safe = jnp.where(owned, local, 0)
    rows = jnp.take(E, safe, axis=0)
    partial = jnp.where(owned[:, None], rows, 0.0).astype(jnp.bfloat16)
    return _allreduce_z(partial).astype(jnp.float32)


# device time: 178781 ns/iter; 2.8053x vs baseline; 2.8053x over previous
import jax
import jax.numpy as jnp
from jax import lax
from jax.experimental import pallas as pl
from jax.experimental.pallas import tpu as pltpu

Z = 4
PLANE = 8
T = 4096
D = 2048
V_SHARD = 8192
GROUP = T // PLANE
ZCHUNK = GROUP // Z
N_ZSTEPS = 2 * (Z - 1)
CW_STEPS = PLANE // 2
CCW_STEPS = PLANE - 1 - CW_STEPS


def _plane_cycle(my_x, my_y):
    p = jnp.where(my_x == 0, my_y, 2 * Z - 1 - my_y)
    succ_x = jnp.where(my_x == 0, (my_y == 3).astype(jnp.int32),
                       (my_y > 0).astype(jnp.int32))
    succ_y = jnp.where(my_x == 0, jnp.where(my_y == 3, 3, my_y + 1),
                       jnp.where(my_y > 0, my_y - 1, 0))
    pred_x = jnp.where(my_x == 0, (my_y == 0).astype(jnp.int32),
                       (my_y < 3).astype(jnp.int32))
    pred_y = jnp.where(my_x == 0, jnp.where(my_y == 0, 0, my_y - 1),
                       jnp.where(my_y == 3, 3, my_y + 1))
    return p, (succ_x, succ_y), (pred_x, pred_y)


def _gather_slab(ids_smem_arr, ids_vmem_arr, E):

    def body(ids_s, ids_v, E_ref, out_ref, gbuf, gsem):
        my_x = lax.axis_index("x")
        my_y = lax.axis_index("y")
        my_z = lax.axis_index("z")
        p, _, _ = _plane_cycle(my_x, my_y)
        off = my_z * V_SHARD
        base = p * GROUP

        def issue(i, cnt):
            idx = ids_s[base + i] - off
            ok = (idx >= 0) & (idx < V_SHARD)

            @pl.when(ok)
            def _():
                pltpu.make_async_copy(
                    E_ref.at[pl.ds(jnp.clip(idx, 0, V_SHARD - 1), 1), :],
                    gbuf.at[pl.ds(i, 1), :],
                    gsem,
                ).start()

            return cnt + ok.astype(jnp.int32)

        cnt = lax.fori_loop(0, GROUP, issue, 0)

        def drain(i, _):
            pltpu.make_async_copy(
                E_ref.at[pl.ds(0, 1), :], gbuf.at[pl.ds(0, 1), :], gsem
            ).wait()
            return 0

        lax.fori_loop(0, cnt, drain, 0)

        my_ids = ids_v[pl.ds(base, GROUP), :]
        owned2d = (my_ids >= off) & (my_ids < off + V_SHARD)
        out_ref[...] = jnp.where(owned2d, gbuf[...], 0.0).astype(jnp.bfloat16)

    return pl.pallas_call(
        body,
        out_shape=jax.ShapeDtypeStruct((GROUP, D), jnp.bfloat16),
        in_specs=[
            pl.BlockSpec(memory_space=pltpu.SMEM),
            pl.BlockSpec(memory_space=pltpu.VMEM),
            pl.BlockSpec(memory_space=pl.ANY),
        ],
        out_specs=pl.BlockSpec(memory_space=pltpu.VMEM),
        scratch_shapes=[
            pltpu.VMEM((GROUP, D), jnp.float32),
            pltpu.SemaphoreType.DMA,
        ],
    )(ids_smem_arr, ids_vmem_arr, E)


def _zring_allreduce(slab):

    def body(in_ref, out_ref, zsend, zrecv, zsend_sems, zrecv_sems, zcredit):
        my_x = lax.axis_index("x")
        my_y = lax.axis_index("y")
        my_z = lax.axis_index("z")
        znext = (my_x, my_y, (my_z + 1) % Z)
        zprev = (my_x, my_y, (my_z - 1) % Z)

        barrier_sem = pltpu.get_barrier_semaphore()
        for nbr in (zprev, znext):
            pl.semaphore_signal(
                barrier_sem, inc=1,
                device_id=nbr, device_id_type=pl.DeviceIdType.MESH,
            )
        pl.semaphore_wait(barrier_sem, 2)

        out_ref[...] = in_ref[...]

        for s in range(N_ZSTEPS):
            slot = s % 2
            if s < Z - 1:
                send_idx = (my_z - s) % Z
            else:
                send_idx = (my_z + 1 - (s - (Z - 1))) % Z
            recv_idx = (send_idx - 1) % Z

            zsend[slot] = out_ref[pl.ds(send_idx * ZCHUNK, ZCHUNK), :]
            if s >= 2:
                pl.semaphore_wait(zcredit, 1)
            rdma = pltpu.make_async_remote_copy(
                src_ref=zsend.at[slot],
                dst_ref=zrecv.at[slot],
                send_sem=zsend_sems.at[slot],
                recv_sem=zrecv_sems.at[slot],
                device_id=znext,
                device_id_type=pl.DeviceIdType.MESH,
            )
            rdma.start()
            rdma.wait()

            if s < Z - 1:
                out_ref[pl.ds(recv_idx * ZCHUNK, ZCHUNK), :] += zrecv[slot]
            else:
                out_ref[pl.ds(recv_idx * ZCHUNK, ZCHUNK), :] = zrecv[slot]
            if s < N_ZSTEPS - 2:
                pl.semaphore_signal(
                    zcredit, inc=1,
                    device_id=zprev, device_id_type=pl.DeviceIdType.MESH,
                )

    return pl.pallas_call(
        body,
        out_shape=jax.ShapeDtypeStruct((GROUP, D), jnp.bfloat16),
        in_specs=[pl.BlockSpec(memory_space=pltpu.VMEM)],
        out_specs=pl.BlockSpec(memory_space=pltpu.VMEM),
        scratch_shapes=[
            pltpu.VMEM((2, ZCHUNK, D), jnp.bfloat16),
            pltpu.VMEM((2, ZCHUNK, D), jnp.bfloat16),
            pltpu.SemaphoreType.DMA((2,)),
            pltpu.SemaphoreType.DMA((2,)),
            pltpu.SemaphoreType.REGULAR,
        ],
        compiler_params=pltpu.CompilerParams(collective_id=0),
    )(slab)


def _plane_allgather(slab):

    def body(in_ref, out_ref, cwbuf, ccwbuf,
             cw_send_sems, cw_recv_sems, cw_credit,
             ccw_send_sems, ccw_recv_sems):
        my_x = lax.axis_index("x")
        my_y = lax.axis_index("y")
        my_z = lax.axis_index("z")
        p, (succ_x, succ_y), (pred_x, pred_y) = _plane_cycle(my_x, my_y)
        succ = (succ_x, succ_y, my_z)
        pred = (pred_x, pred_y, my_z)

        barrier_sem = pltpu.get_barrier_semaphore()
        for nbr in (pred, succ):
            pl.semaphore_signal(
                barrier_sem, inc=1,
                device_id=nbr, device_id_type=pl.DeviceIdType.MESH,
            )
        pl.semaphore_wait(barrier_sem, 2)

        out_ref[pl.ds(p * GROUP, GROUP), :] = in_ref[...]

        for k in range(CW_STEPS):
            slot = k % 3
            cw_src = in_ref if k == 0 else cwbuf.at[(k - 1) % 3]
            if k >= 3:
                pl.semaphore_wait(cw_credit, 1)
            cw = pltpu.make_async_remote_copy(
                src_ref=cw_src,
                dst_ref=cwbuf.at[slot],
                send_sem=cw_send_sems.at[slot],
                recv_sem=cw_recv_sems.at[slot],
                device_id=succ,
                device_id_type=pl.DeviceIdType.MESH,
            )
            cw.start()

            if k < CCW_STEPS:
                ccw_src = in_ref if k == 0 else ccwbuf.at[k - 1]
                ccw = pltpu.make_async_remote_copy(
                    src_ref=ccw_src,
                    dst_ref=ccwbuf.at[k],
                    send_sem=ccw_send_sems.at[k],
                    recv_sem=ccw_recv_sems.at[k],
                    device_id=pred,
                    device_id_type=pl.DeviceIdType.MESH,
                )
                ccw.start()
                ccw.wait()
                o_ccw = (p + 1 + k) % PLANE
                out_ref[pl.ds(o_ccw * GROUP, GROUP), :] = ccwbuf[k]

            cw.wait()
            o_cw = (p - 1 - k) % PLANE
            out_ref[pl.ds(o_cw * GROUP, GROUP), :] = cwbuf[slot]
            if k == 1:
                pl.semaphore_signal(
                    cw_credit, inc=1,
                    device_id=pred, device_id_type=pl.DeviceIdType.MESH,
                )

    return pl.pallas_call(
        body,
        out_shape=jax.ShapeDtypeStruct((T, D), jnp.bfloat16),
        in_specs=[pl.BlockSpec(memory_space=pltpu.VMEM)],
        out_specs=pl.BlockSpec(memory_space=pltpu.VMEM),
        scratch_shapes=[
            pltpu.VMEM((3, GROUP, D), jnp.bfloat16),
            pltpu.VMEM((3, GROUP, D), jnp.bfloat16),
            pltpu.SemaphoreType.DMA((3,)),
            pltpu.SemaphoreType.DMA((3,)),
            pltpu.SemaphoreType.REGULAR,
            pltpu.SemaphoreType.DMA((3,)),
            pltpu.SemaphoreType.DMA((3,)),
        ],
        compiler_params=pltpu.CompilerParams(collective_id=1),
    )(slab)


def kernel(ids, E):
    slab = _gather_slab(ids, ids.reshape(T, 1), E)
    slab = _zring_allreduce(slab)
    full = _plane_allgather(slab)
    return full.astype(jnp.float32)


# device time: 166867 ns/iter; 3.0056x vs baseline; 1.0714x over previous
import jax
import jax.numpy as jnp
from jax import lax
from jax.experimental import pallas as pl
from jax.experimental.pallas import tpu as pltpu

Z = 4
PLANE = 8
T = 4096
D = 2048
V_SHARD = 8192
GROUP = T // PLANE
ZCHUNK = GROUP // Z
N_ZSTEPS = 2 * (Z - 1)
CW_STEPS = PLANE // 2
CCW_STEPS = PLANE - 1 - CW_STEPS


def _plane_cycle(my_x, my_y):
    p = jnp.where(my_x == 0, my_y, 2 * Z - 1 - my_y)
    succ_x = jnp.where(my_x == 0, (my_y == 3).astype(jnp.int32),
                       (my_y > 0).astype(jnp.int32))
    succ_y = jnp.where(my_x == 0, jnp.where(my_y == 3, 3, my_y + 1),
                       jnp.where(my_y > 0, my_y - 1, 0))
    pred_x = jnp.where(my_x == 0, (my_y == 0).astype(jnp.int32),
                       (my_y < 3).astype(jnp.int32))
    pred_y = jnp.where(my_x == 0, jnp.where(my_y == 0, 0, my_y - 1),
                       jnp.where(my_y == 3, 3, my_y + 1))
    return p, (succ_x, succ_y), (pred_x, pred_y)


def _gather_slab(ids_smem_arr, ids_vmem_arr, E):

    def body(ids_s, ids_v, E_ref, out_ref, gbuf, gsem):
        my_x = lax.axis_index("x")
        my_y = lax.axis_index("y")
        my_z = lax.axis_index("z")
        p, _, _ = _plane_cycle(my_x, my_y)
        off = my_z * V_SHARD
        base = p * GROUP

        def issue(i, cnt):
            idx = ids_s[base + i] - off
            ok = (idx >= 0) & (idx < V_SHARD)

            @pl.when(ok)
            def _():
                pltpu.make_async_copy(
                    E_ref.at[pl.ds(jnp.clip(idx, 0, V_SHARD - 1), 1), :],
                    gbuf.at[pl.ds(i, 1), :],
                    gsem,
                ).start()

            return cnt + ok.astype(jnp.int32)

        cnt = lax.fori_loop(0, GROUP, issue, 0)

        def drain(i, _):
            pltpu.make_async_copy(
                E_ref.at[pl.ds(0, 1), :], gbuf.at[pl.ds(0, 1), :], gsem
            ).wait()
            return 0

        lax.fori_loop(0, cnt, drain, 0)

        my_ids = ids_v[pl.ds(base, GROUP), :]
        owned2d = (my_ids >= off) & (my_ids < off + V_SHARD)
        out_ref[...] = jnp.where(owned2d, gbuf[...], 0.0).astype(jnp.bfloat16)

    return pl.pallas_call(
        body,
        out_shape=jax.ShapeDtypeStruct((GROUP, D), jnp.bfloat16),
        in_specs=[
            pl.BlockSpec(memory_space=pltpu.SMEM),
            pl.BlockSpec(memory_space=pltpu.VMEM),
            pl.BlockSpec(memory_space=pl.ANY),
        ],
        out_specs=pl.BlockSpec(memory_space=pltpu.VMEM),
        scratch_shapes=[
            pltpu.VMEM((GROUP, D), jnp.float32),
            pltpu.SemaphoreType.DMA,
        ],
    )(ids_smem_arr, ids_vmem_arr, E)


def _zring_allreduce(slab):

    def body(in_ref, out_ref, zsend, zrecv, zsend_sems, zrecv_sems, zcredit):
        my_x = lax.axis_index("x")
        my_y = lax.axis_index("y")
        my_z = lax.axis_index("z")
        znext = (my_x, my_y, (my_z + 1) % Z)
        zprev = (my_x, my_y, (my_z - 1) % Z)

        barrier_sem = pltpu.get_barrier_semaphore()
        for nbr in (zprev, znext):
            pl.semaphore_signal(
                barrier_sem, inc=1,
                device_id=nbr, device_id_type=pl.DeviceIdType.MESH,
            )
        pl.semaphore_wait(barrier_sem, 2)

        out_ref[...] = in_ref[...]

        for s in range(N_ZSTEPS):
            slot = s % 2
            if s < Z - 1:
                send_idx = (my_z - s) % Z
            else:
                send_idx = (my_z + 1 - (s - (Z - 1))) % Z
            recv_idx = (send_idx - 1) % Z

            zsend[slot] = out_ref[pl.ds(send_idx * ZCHUNK, ZCHUNK), :]
            if s >= 2:
                pl.semaphore_wait(zcredit, 1)
            rdma = pltpu.make_async_remote_copy(
                src_ref=zsend.at[slot],
                dst_ref=zrecv.at[slot],
                send_sem=zsend_sems.at[slot],
                recv_sem=zrecv_sems.at[slot],
                device_id=znext,
                device_id_type=pl.DeviceIdType.MESH,
            )
            rdma.start()
            rdma.wait()

            if s < Z - 1:
                out_ref[pl.ds(recv_idx * ZCHUNK, ZCHUNK), :] += zrecv[slot]
            else:
                out_ref[pl.ds(recv_idx * ZCHUNK, ZCHUNK), :] = zrecv[slot]
            if s < N_ZSTEPS - 2:
                pl.semaphore_signal(
                    zcredit, inc=1,
                    device_id=zprev, device_id_type=pl.DeviceIdType.MESH,
                )

    return pl.pallas_call(
        body,
        out_shape=jax.ShapeDtypeStruct((GROUP, D), jnp.bfloat16),
        in_specs=[pl.BlockSpec(memory_space=pltpu.VMEM)],
        out_specs=pl.BlockSpec(memory_space=pltpu.VMEM),
        scratch_shapes=[
            pltpu.VMEM((2, ZCHUNK, D), jnp.bfloat16),
            pltpu.VMEM((2, ZCHUNK, D), jnp.bfloat16),
            pltpu.SemaphoreType.DMA((2,)),
            pltpu.SemaphoreType.DMA((2,)),
            pltpu.SemaphoreType.REGULAR,
        ],
        compiler_params=pltpu.CompilerParams(collective_id=0),
    )(slab)


def _plane_allgather(slab):

    def body(in_ref, out_ref, cwbuf, ccwbuf,
             cw_send_sems, cw_recv_sems, cw_credit,
             ccw_send_sems, ccw_recv_sems):
        my_x = lax.axis_index("x")
        my_y = lax.axis_index("y")
        my_z = lax.axis_index("z")
        p, (succ_x, succ_y), (pred_x, pred_y) = _plane_cycle(my_x, my_y)
        succ = (succ_x, succ_y, my_z)
        pred = (pred_x, pred_y, my_z)

        barrier_sem = pltpu.get_barrier_semaphore()
        for nbr in (pred, succ):
            pl.semaphore_signal(
                barrier_sem, inc=1,
                device_id=nbr, device_id_type=pl.DeviceIdType.MESH,
            )
        pl.semaphore_wait(barrier_sem, 2)

        out_ref[pl.ds(p * GROUP, GROUP), :] = in_ref[...]

        HALF = GROUP // 2
        for k in range(CW_STEPS):
            slot = k % 3
            if k >= 3:
                pl.semaphore_wait(cw_credit, 1)
            if k == 0:
                cw_src, cw_dst = in_ref, cwbuf.at[slot]
            elif k < 3:
                cw_src, cw_dst = cwbuf.at[k - 1], cwbuf.at[slot]
            else:
                cw_src = cwbuf.at[2, pl.ds(0, HALF), :]
                cw_dst = cwbuf.at[0, pl.ds(0, HALF), :]
            cw = pltpu.make_async_remote_copy(
                src_ref=cw_src,
                dst_ref=cw_dst,
                send_sem=cw_send_sems.at[slot],
                recv_sem=cw_recv_sems.at[slot],
                device_id=succ,
                device_id_type=pl.DeviceIdType.MESH,
            )
            cw.start()

            if k == 0:
                ccw_src, ccw_dst = in_ref, ccwbuf.at[k]
            elif k < 3:
                ccw_src, ccw_dst = ccwbuf.at[k - 1], ccwbuf.at[k]
            else:
                ccw_src = ccwbuf.at[2, pl.ds(HALF, HALF), :]
                ccw_dst = cwbuf.at[0, pl.ds(HALF, HALF), :]
            ccw = pltpu.make_async_remote_copy(
                src_ref=ccw_src,
                dst_ref=ccw_dst,
                send_sem=ccw_send_sems.at[k],
                recv_sem=ccw_recv_sems.at[k],
                device_id=pred,
                device_id_type=pl.DeviceIdType.MESH,
            )
            ccw.start()

            if k >= 1:
                out_ref[pl.ds(((p - k) % PLANE) * GROUP, GROUP), :] = (
                    cwbuf[k - 1])
                out_ref[pl.ds(((p + k) % PLANE) * GROUP, GROUP), :] = (
                    ccwbuf[k - 1])

            ccw.wait()
            cw.wait()
            if k == 1:
                pl.semaphore_signal(
                    cw_credit, inc=1,
                    device_id=pred, device_id_type=pl.DeviceIdType.MESH,
                )

        out_ref[pl.ds(((p - 4) % PLANE) * GROUP, GROUP), :] = cwbuf[0]

    return pl.pallas_call(
        body,
        out_shape=jax.ShapeDtypeStruct((T, D), jnp.bfloat16),
        in_specs=[pl.BlockSpec(memory_space=pltpu.VMEM)],
        out_specs=pl.BlockSpec(memory_space=pltpu.VMEM),
        scratch_shapes=[
            pltpu.VMEM((3, GROUP, D), jnp.bfloat16),
            pltpu.VMEM((3, GROUP, D), jnp.bfloat16),
            pltpu.SemaphoreType.DMA((3,)),
            pltpu.SemaphoreType.DMA((3,)),
            pltpu.SemaphoreType.REGULAR,
            pltpu.SemaphoreType.DMA((4,)),
            pltpu.SemaphoreType.DMA((4,)),
        ],
        compiler_params=pltpu.CompilerParams(collective_id=1),
    )(slab)


def kernel(ids, E):
    slab = _gather_slab(ids, ids.reshape(T, 1), E)
    slab = _zring_allreduce(slab)
    full = _plane_allgather(slab)
    return full.astype(jnp.float32)
